# Initial kernel scaffold; baseline (speedup 1.0000x reference)
import jax
import jax.numpy as jnp
from jax.experimental import pallas as pl


def kernel(x, edge_index, W1, b1, W2, b2):
    raise NotImplementedError("write your pallas kernel here")


# speedup vs baseline: 19.2930x; 19.2930x over previous
import functools

import jax
import jax.numpy as jnp
from jax import lax
from jax.experimental import pallas as pl
from jax.experimental.pallas import tpu as pltpu
from jax.experimental.pallas import tpu_sc as plsc

N_NODES = 10000
D_FEAT = 128
N_HID = 128
N_EDGES = 320000

NC = 2
NS = 16
EPT = N_EDGES // (NC * NS)
K = 80
CH = EPT // K
NROWS = 10240
RPT = NROWS // NS
NPAD = 16384
DPT = NPAD // NS

_mesh = plsc.VectorSubcoreMesh(core_axis_name="c", subcore_axis_name="s")


@functools.partial(
    pl.kernel,
    out_type=jax.ShapeDtypeStruct((NC * NPAD,), jnp.float32),
    mesh=_mesh,
    scratch_types=dict(
        deg_sh=pltpu.VMEM_SHARED((NPAD,), jnp.float32),
        dstb=pltpu.VMEM((CH, K), jnp.int32),
        ones=pltpu.VMEM((K,), jnp.float32),
        zv=pltpu.VMEM((DPT,), jnp.float32),
    ),
)
def _sc_deg(dst_hbm, deg_out, *, deg_sh, dstb, ones, zv):
    c = lax.axis_index("c")
    t = lax.axis_index("s")
    pltpu.sync_copy(dst_hbm.at[c, t], dstb)
    for i in range(K // 16):
        ones[pl.ds(i * 16, 16)] = jnp.ones((16,), jnp.float32)
    for i in range(DPT // 16):
        zv[pl.ds(i * 16, 16)] = jnp.zeros((16,), jnp.float32)
    pltpu.sync_copy(zv, deg_sh.at[pl.ds(t * DPT, DPT)])
    plsc.subcore_barrier()

    def chunk(j, carry):
        pltpu.sync_copy(ones, deg_sh.at[dstb.at[j]], add=True)
        return carry

    lax.fori_loop(0, CH, chunk, 0)
    plsc.subcore_barrier()
    pltpu.sync_copy(
        deg_sh.at[pl.ds(t * DPT, DPT)],
        deg_out.at[pl.ds(c * NPAD + t * DPT, DPT)],
    )


@functools.partial(
    pl.kernel,
    out_type=jax.ShapeDtypeStruct((NC, NROWS, N_HID), jnp.float32),
    mesh=_mesh,
    scratch_types=dict(
        acc_sh=pltpu.VMEM_SHARED((NROWS, N_HID), jnp.float32),
        srcb=pltpu.VMEM((CH, K), jnp.int32),
        dstb=pltpu.VMEM((CH, K), jnp.int32),
        rows=pltpu.VMEM((K, N_HID), jnp.float32),
        zv=pltpu.VMEM((8, N_HID), jnp.float32),
    ),
)
def _sc_scatter(src_hbm, dst_hbm, g_hbm, out_hbm, *, acc_sh, srcb, dstb, rows, zv):
    c = lax.axis_index("c")
    t = lax.axis_index("s")
    pltpu.sync_copy(src_hbm.at[c, t], srcb)
    pltpu.sync_copy(dst_hbm.at[c, t], dstb)

    for i in range(8):
        for j in range(N_HID // 16):
            zv[i, pl.ds(j * 16, 16)] = jnp.zeros((16,), jnp.float32)

    def zcopy(i, carry):
        pltpu.sync_copy(zv, acc_sh.at[pl.ds(t * RPT + i * 8, 8)])
        return carry

    lax.fori_loop(0, RPT // 8, zcopy, 0)
    plsc.subcore_barrier()

    def chunk(j, carry):
        pltpu.sync_copy(g_hbm.at[srcb.at[j]], rows)
        pltpu.sync_copy(rows, acc_sh.at[dstb.at[j]], add=True)
        return carry

    lax.fori_loop(0, CH, chunk, 0)
    plsc.subcore_barrier()
    pltpu.sync_copy(
        acc_sh.at[pl.ds(t * RPT, RPT)],
        out_hbm.at[c, pl.ds(t * RPT, RPT)],
    )


_RB = 400
_GRID = N_NODES // _RB


def _tc_g1_body(x_ref, w_ref, d0_ref, d1_ref, g_ref, dinv_ref):
    deg = d0_ref[...] + d1_ref[...] + 1.0
    dv = lax.rsqrt(deg)
    dinv_ref[...] = dv
    h = jnp.dot(x_ref[...], w_ref[...], preferred_element_type=jnp.float32)
    g_ref[...] = h * dv


def _tc_g1(x, W1, deg0, deg1):
    return pl.pallas_call(
        _tc_g1_body,
        grid=(_GRID,),
        in_specs=[
            pl.BlockSpec((_RB, D_FEAT), lambda i: (i, 0)),
            pl.BlockSpec((D_FEAT, N_HID), lambda i: (0, 0)),
            pl.BlockSpec((_RB, 1), lambda i: (i, 0)),
            pl.BlockSpec((_RB, 1), lambda i: (i, 0)),
        ],
        out_specs=[
            pl.BlockSpec((_RB, N_HID), lambda i: (i, 0)),
            pl.BlockSpec((_RB, 1), lambda i: (i, 0)),
        ],
        out_shape=[
            jax.ShapeDtypeStruct((N_NODES, N_HID), jnp.float32),
            jax.ShapeDtypeStruct((N_NODES, 1), jnp.float32),
        ],
    )(x, W1, deg0, deg1)


def _tc_mid_body(s_ref, g_ref, dv_ref, b_ref, w_ref, g2_ref):
    dv = dv_ref[...]
    z = jnp.maximum((s_ref[0] + s_ref[1] + g_ref[...]) * dv + b_ref[...], 0.0)
    h2 = jnp.dot(z, w_ref[...], preferred_element_type=jnp.float32)
    g2_ref[...] = h2 * dv


def _tc_mid(s, g1, dinv, b1, W2):
    return pl.pallas_call(
        _tc_mid_body,
        grid=(_GRID,),
        in_specs=[
            pl.BlockSpec((NC, _RB, N_HID), lambda i: (0, i, 0)),
            pl.BlockSpec((_RB, N_HID), lambda i: (i, 0)),
            pl.BlockSpec((_RB, 1), lambda i: (i, 0)),
            pl.BlockSpec((1, N_HID), lambda i: (0, 0)),
            pl.BlockSpec((N_HID, N_HID), lambda i: (0, 0)),
        ],
        out_specs=pl.BlockSpec((_RB, N_HID), lambda i: (i, 0)),
        out_shape=jax.ShapeDtypeStruct((N_NODES, N_HID), jnp.float32),
    )(s, g1, dinv, b1, W2)


def _tc_out_body(s_ref, g_ref, dv_ref, b_ref, o_ref):
    o_ref[...] = jnp.maximum(
        (s_ref[0] + s_ref[1] + g_ref[...]) * dv_ref[...] + b_ref[...], 0.0
    )


def _tc_out(s, g2, dinv, b2):
    return pl.pallas_call(
        _tc_out_body,
        grid=(_GRID,),
        in_specs=[
            pl.BlockSpec((NC, _RB, N_HID), lambda i: (0, i, 0)),
            pl.BlockSpec((_RB, N_HID), lambda i: (i, 0)),
            pl.BlockSpec((_RB, 1), lambda i: (i, 0)),
            pl.BlockSpec((1, N_HID), lambda i: (0, 0)),
        ],
        out_specs=pl.BlockSpec((_RB, N_HID), lambda i: (i, 0)),
        out_shape=jax.ShapeDtypeStruct((N_NODES, N_HID), jnp.float32),
    )(s, g2, dinv, b2)


@jax.jit
def kernel(x, edge_index, W1, b1, W2, b2):
    src = edge_index[0].astype(jnp.int32).reshape(NC, NS, CH, K)
    dst = edge_index[1].astype(jnp.int32).reshape(NC, NS, CH, K)

    degp = _sc_deg(dst).reshape(NC, NPAD)
    deg0 = degp[0, :N_NODES].reshape(N_NODES, 1)
    deg1 = degp[1, :N_NODES].reshape(N_NODES, 1)

    g1, dinv = _tc_g1(x, W1, deg0, deg1)

    s = _sc_scatter(src, dst, g1)
    g2 = _tc_mid(s, g1, dinv, b1.reshape(1, N_HID), W2)

    s2 = _sc_scatter(src, dst, g2)
    return _tc_out(s2, g2, dinv, b2.reshape(1, N_HID))


# speedup vs baseline: 28.8900x; 1.4974x over previous
import functools

import jax
import jax.numpy as jnp
from jax import lax
from jax.experimental import pallas as pl
from jax.experimental.pallas import tpu as pltpu
from jax.experimental.pallas import tpu_sc as plsc

N_NODES = 10000
D_FEAT = 128
N_HID = 128
N_EDGES = 320000

NC = 2
NS = 16
EPT = N_EDGES // (NC * NS)
K = 80
CH = EPT // K
NROWS = 10240
RPT = NROWS // NS
NPAD = 16384
DPT = NPAD // NS

_mesh = plsc.VectorSubcoreMesh(core_axis_name="c", subcore_axis_name="s")


@functools.partial(
    pl.kernel,
    out_type=jax.ShapeDtypeStruct((NC * NPAD,), jnp.float32),
    mesh=_mesh,
    scratch_types=dict(
        deg_sh=pltpu.VMEM_SHARED((NPAD,), jnp.float32),
        dstb=pltpu.VMEM((CH, K), jnp.int32),
        ones=pltpu.VMEM((K,), jnp.float32),
        zv=pltpu.VMEM((DPT,), jnp.float32),
    ),
)
def _sc_deg(dst_hbm, deg_out, *, deg_sh, dstb, ones, zv):
    c = lax.axis_index("c")
    t = lax.axis_index("s")
    pltpu.sync_copy(dst_hbm.at[c, t], dstb)
    for i in range(K // 16):
        ones[pl.ds(i * 16, 16)] = jnp.ones((16,), jnp.float32)
    for i in range(DPT // 16):
        zv[pl.ds(i * 16, 16)] = jnp.zeros((16,), jnp.float32)
    pltpu.sync_copy(zv, deg_sh.at[pl.ds(t * DPT, DPT)])
    plsc.subcore_barrier()

    def chunk(j, carry):
        pltpu.sync_copy(ones, deg_sh.at[dstb.at[j]], add=True)
        return carry

    lax.fori_loop(0, CH, chunk, 0)
    plsc.subcore_barrier()
    pltpu.sync_copy(
        deg_sh.at[pl.ds(t * DPT, DPT)],
        deg_out.at[pl.ds(c * NPAD + t * DPT, DPT)],
    )


@functools.partial(
    pl.kernel,
    out_type=jax.ShapeDtypeStruct((NC, NROWS, N_HID), jnp.float32),
    mesh=_mesh,
    scratch_types=dict(
        acc_sh=pltpu.VMEM_SHARED((NROWS, N_HID), jnp.float32),
        dstb=pltpu.VMEM((CH, K), jnp.int32),
        ib0=pltpu.VMEM((K,), jnp.int32),
        ib1=pltpu.VMEM((K,), jnp.int32),
        rows0=pltpu.VMEM((K, N_HID), jnp.float32),
        rows1=pltpu.VMEM((K, N_HID), jnp.float32),
        sem0=pltpu.SemaphoreType.DMA,
        sem1=pltpu.SemaphoreType.DMA,
        semi0=pltpu.SemaphoreType.DMA,
        semi1=pltpu.SemaphoreType.DMA,
    ),
)
def _sc_scatter(src_hbm, dst_hbm, g_hbm, out_hbm, *, acc_sh, dstb, ib0, ib1,
                rows0, rows1, sem0, sem1, semi0, semi1):
    c = lax.axis_index("c")
    t = lax.axis_index("s")

    def zrow(i, carry):
        for j in range(N_HID // 16):
            rows0[i, pl.ds(j * 16, 16)] = jnp.zeros((16,), jnp.float32)
        return carry

    lax.fori_loop(0, K, zrow, 0)
    for i in range(RPT // K):
        pltpu.async_copy(rows0, acc_sh.at[pl.ds(t * RPT + i * K, K)], sem0)
    pltpu.sync_copy(dst_hbm.at[c, t], dstb)
    for i in range(RPT // K):
        pltpu.make_async_copy(rows0, acc_sh.at[pl.ds(t * RPT, K)], sem0).wait()
    plsc.subcore_barrier()

    pltpu.sync_copy(src_hbm.at[c, t, 0], ib0)
    pltpu.sync_copy(src_hbm.at[c, t, 1], ib1)
    pltpu.async_copy(g_hbm.at[ib0], rows0, sem0)
    pltpu.async_copy(g_hbm.at[ib1], rows1, sem1)

    def pair(i, carry):
        j0 = 2 * i
        pltpu.make_async_copy(g_hbm.at[ib0], rows0, sem0).wait()

        @pl.when(j0 + 2 < CH)
        def _():
            pltpu.async_copy(src_hbm.at[c, t, j0 + 2], ib0, semi0)

        pltpu.sync_copy(rows0, acc_sh.at[dstb.at[j0]], add=True)

        @pl.when(j0 + 2 < CH)
        def _():
            pltpu.make_async_copy(src_hbm.at[c, t, 0], ib0, semi0).wait()
            pltpu.async_copy(g_hbm.at[ib0], rows0, sem0)

        pltpu.make_async_copy(g_hbm.at[ib1], rows1, sem1).wait()

        @pl.when(j0 + 3 < CH)
        def _():
            pltpu.async_copy(src_hbm.at[c, t, j0 + 3], ib1, semi1)

        pltpu.sync_copy(rows1, acc_sh.at[dstb.at[j0 + 1]], add=True)

        @pl.when(j0 + 3 < CH)
        def _():
            pltpu.make_async_copy(src_hbm.at[c, t, 0], ib1, semi1).wait()
            pltpu.async_copy(g_hbm.at[ib1], rows1, sem1)

        return carry

    lax.fori_loop(0, CH // 2, pair, 0)
    if CH % 2:
        pltpu.make_async_copy(g_hbm.at[ib0], rows0, sem0).wait()
        pltpu.sync_copy(rows0, acc_sh.at[dstb.at[CH - 1]], add=True)

    plsc.subcore_barrier()
    pltpu.sync_copy(
        acc_sh.at[pl.ds(t * RPT, RPT)],
        out_hbm.at[c, pl.ds(t * RPT, RPT)],
    )


_RB = 400
_GRID = N_NODES // _RB


def _tc_g1_body(x_ref, w_ref, d0_ref, d1_ref, g_ref, dinv_ref):
    deg = d0_ref[...] + d1_ref[...] + 1.0
    dv = lax.rsqrt(deg)
    dinv_ref[...] = dv
    h = jnp.dot(x_ref[...], w_ref[...], preferred_element_type=jnp.float32)
    g_ref[...] = h * dv


def _tc_g1(x, W1, deg0, deg1):
    return pl.pallas_call(
        _tc_g1_body,
        grid=(_GRID,),
        in_specs=[
            pl.BlockSpec((_RB, D_FEAT), lambda i: (i, 0)),
            pl.BlockSpec((D_FEAT, N_HID), lambda i: (0, 0)),
            pl.BlockSpec((_RB, 1), lambda i: (i, 0)),
            pl.BlockSpec((_RB, 1), lambda i: (i, 0)),
        ],
        out_specs=[
            pl.BlockSpec((_RB, N_HID), lambda i: (i, 0)),
            pl.BlockSpec((_RB, 1), lambda i: (i, 0)),
        ],
        out_shape=[
            jax.ShapeDtypeStruct((N_NODES, N_HID), jnp.float32),
            jax.ShapeDtypeStruct((N_NODES, 1), jnp.float32),
        ],
    )(x, W1, deg0, deg1)


def _tc_mid_body(s_ref, g_ref, dv_ref, b_ref, w_ref, g2_ref):
    dv = dv_ref[...]
    z = jnp.maximum((s_ref[0] + s_ref[1] + g_ref[...]) * dv + b_ref[...], 0.0)
    h2 = jnp.dot(z, w_ref[...], preferred_element_type=jnp.float32)
    g2_ref[...] = h2 * dv


def _tc_mid(s, g1, dinv, b1, W2):
    return pl.pallas_call(
        _tc_mid_body,
        grid=(_GRID,),
        in_specs=[
            pl.BlockSpec((NC, _RB, N_HID), lambda i: (0, i, 0)),
            pl.BlockSpec((_RB, N_HID), lambda i: (i, 0)),
            pl.BlockSpec((_RB, 1), lambda i: (i, 0)),
            pl.BlockSpec((1, N_HID), lambda i: (0, 0)),
            pl.BlockSpec((N_HID, N_HID), lambda i: (0, 0)),
        ],
        out_specs=pl.BlockSpec((_RB, N_HID), lambda i: (i, 0)),
        out_shape=jax.ShapeDtypeStruct((N_NODES, N_HID), jnp.float32),
    )(s, g1, dinv, b1, W2)


def _tc_out_body(s_ref, g_ref, dv_ref, b_ref, o_ref):
    o_ref[...] = jnp.maximum(
        (s_ref[0] + s_ref[1] + g_ref[...]) * dv_ref[...] + b_ref[...], 0.0
    )


def _tc_out(s, g2, dinv, b2):
    return pl.pallas_call(
        _tc_out_body,
        grid=(_GRID,),
        in_specs=[
            pl.BlockSpec((NC, _RB, N_HID), lambda i: (0, i, 0)),
            pl.BlockSpec((_RB, N_HID), lambda i: (i, 0)),
            pl.BlockSpec((_RB, 1), lambda i: (i, 0)),
            pl.BlockSpec((1, N_HID), lambda i: (0, 0)),
        ],
        out_specs=pl.BlockSpec((_RB, N_HID), lambda i: (i, 0)),
        out_shape=jax.ShapeDtypeStruct((N_NODES, N_HID), jnp.float32),
    )(s, g2, dinv, b2)


@jax.jit
def kernel(x, edge_index, W1, b1, W2, b2):
    src = edge_index[0].astype(jnp.int32).reshape(NC, NS, CH, K)
    dst = edge_index[1].astype(jnp.int32).reshape(NC, NS, CH, K)

    degp = _sc_deg(dst).reshape(NC, NPAD)
    deg0 = degp[0, :N_NODES].reshape(N_NODES, 1)
    deg1 = degp[1, :N_NODES].reshape(N_NODES, 1)

    g1, dinv = _tc_g1(x, W1, deg0, deg1)

    s = _sc_scatter(src, dst, g1)
    g2 = _tc_mid(s, g1, dinv, b1.reshape(1, N_HID), W2)

    s2 = _sc_scatter(src, dst, g2)
    return _tc_out(s2, g2, dinv, b2.reshape(1, N_HID))


# speedup vs baseline: 32.1200x; 1.1118x over previous
import functools

import jax
import jax.numpy as jnp
from jax import lax
from jax.experimental import pallas as pl
from jax.experimental.pallas import tpu as pltpu
from jax.experimental.pallas import tpu_sc as plsc

N_NODES = 10000
D_FEAT = 128
N_HID = 128
N_EDGES = 320000

NC = 2
NS = 16
K = 128
CH = 80
EPAD = NC * NS * CH * K
NROWS = 10112
RPT = NROWS // NS
NPAD = 16384
DPT = NPAD // NS

_mesh = plsc.VectorSubcoreMesh(core_axis_name="c", subcore_axis_name="s")


@functools.partial(
    pl.kernel,
    out_type=jax.ShapeDtypeStruct((NC * NPAD,), jnp.float32),
    mesh=_mesh,
    scratch_types=dict(
        deg_sh=pltpu.VMEM_SHARED((NPAD,), jnp.float32),
        dstb=pltpu.VMEM((CH, K), jnp.int32),
        ones=pltpu.VMEM((K,), jnp.float32),
        zv=pltpu.VMEM((DPT,), jnp.float32),
    ),
)
def _sc_deg(dst_hbm, deg_out, *, deg_sh, dstb, ones, zv):
    c = lax.axis_index("c")
    t = lax.axis_index("s")
    pltpu.sync_copy(dst_hbm.at[c, t], dstb)
    for i in range(K // 16):
        ones[pl.ds(i * 16, 16)] = jnp.ones((16,), jnp.float32)
    for i in range(DPT // 16):
        zv[pl.ds(i * 16, 16)] = jnp.zeros((16,), jnp.float32)
    pltpu.sync_copy(zv, deg_sh.at[pl.ds(t * DPT, DPT)])
    plsc.subcore_barrier()

    def chunk(j, carry):
        pltpu.sync_copy(ones, deg_sh.at[dstb.at[j]], add=True)
        return carry

    lax.fori_loop(0, CH, chunk, 0)
    plsc.subcore_barrier()
    pltpu.sync_copy(
        deg_sh.at[pl.ds(t * DPT, DPT)],
        deg_out.at[pl.ds(c * NPAD + t * DPT, DPT)],
    )


@functools.partial(
    pl.kernel,
    out_type=jax.ShapeDtypeStruct((NC, NROWS, N_HID), jnp.float32),
    mesh=_mesh,
    scratch_types=dict(
        acc_sh=pltpu.VMEM_SHARED((NROWS, N_HID), jnp.float32),
        ib0s=pltpu.VMEM((K,), jnp.int32),
        ib1s=pltpu.VMEM((K,), jnp.int32),
        ib0d=pltpu.VMEM((K,), jnp.int32),
        ib1d=pltpu.VMEM((K,), jnp.int32),
        rows0=pltpu.VMEM((K, N_HID), jnp.float32),
        rows1=pltpu.VMEM((K, N_HID), jnp.float32),
        sem0=pltpu.SemaphoreType.DMA,
        sem1=pltpu.SemaphoreType.DMA,
        semi0s=pltpu.SemaphoreType.DMA,
        semi1s=pltpu.SemaphoreType.DMA,
        semi0d=pltpu.SemaphoreType.DMA,
        semi1d=pltpu.SemaphoreType.DMA,
    ),
)
def _sc_scatter(src_hbm, dst_hbm, g_hbm, out_hbm, *, acc_sh, ib0s, ib1s,
                ib0d, ib1d, rows0, rows1, sem0, sem1, semi0s, semi1s,
                semi0d, semi1d):
    c = lax.axis_index("c")
    t = lax.axis_index("s")

    def zrow(i, carry):
        for j in range(N_HID // 16):
            rows0[i, pl.ds(j * 16, 16)] = jnp.zeros((16,), jnp.float32)
        return carry

    lax.fori_loop(0, K, zrow, 0)
    pltpu.async_copy(rows0, acc_sh.at[pl.ds(t * RPT, K)], sem0)
    pltpu.async_copy(rows0, acc_sh.at[pl.ds(t * RPT + K, K)], sem0)
    pltpu.async_copy(rows0, acc_sh.at[pl.ds(t * RPT + 2 * K, K)], sem0)
    pltpu.async_copy(rows0, acc_sh.at[pl.ds(t * RPT + 3 * K, K)], sem0)
    pltpu.async_copy(
        rows0.at[pl.ds(0, RPT - 4 * K)],
        acc_sh.at[pl.ds(t * RPT + 4 * K, RPT - 4 * K)], sem1)
    for _ in range(4):
        pltpu.make_async_copy(rows0, acc_sh.at[pl.ds(t * RPT, K)], sem0).wait()
    pltpu.make_async_copy(
        rows0.at[pl.ds(0, RPT - 4 * K)],
        acc_sh.at[pl.ds(t * RPT, RPT - 4 * K)], sem1).wait()
    plsc.subcore_barrier()

    pltpu.async_copy(src_hbm.at[c, t, 0], ib0s, semi0s)
    pltpu.async_copy(src_hbm.at[c, t, 1], ib1s, semi1s)
    pltpu.async_copy(dst_hbm.at[c, t, 0], ib0d, semi0d)
    pltpu.async_copy(dst_hbm.at[c, t, 1], ib1d, semi1d)
    pltpu.make_async_copy(src_hbm.at[c, t, 0], ib0s, semi0s).wait()
    pltpu.async_copy(g_hbm.at[ib0s], rows0, sem0)
    pltpu.make_async_copy(src_hbm.at[c, t, 1], ib1s, semi1s).wait()
    pltpu.async_copy(g_hbm.at[ib1s], rows1, sem1)

    def pair(i, carry):
        j0 = 2 * i
        pltpu.make_async_copy(g_hbm.at[ib0s], rows0, sem0).wait()
        pltpu.make_async_copy(dst_hbm.at[c, t, 0], ib0d, semi0d).wait()

        @pl.when(j0 + 2 < CH)
        def _():
            pltpu.async_copy(src_hbm.at[c, t, j0 + 2], ib0s, semi0s)

        pltpu.sync_copy(rows0, acc_sh.at[ib0d], add=True)

        @pl.when(j0 + 2 < CH)
        def _():
            pltpu.async_copy(dst_hbm.at[c, t, j0 + 2], ib0d, semi0d)
            pltpu.make_async_copy(src_hbm.at[c, t, 0], ib0s, semi0s).wait()
            pltpu.async_copy(g_hbm.at[ib0s], rows0, sem0)

        pltpu.make_async_copy(g_hbm.at[ib1s], rows1, sem1).wait()
        pltpu.make_async_copy(dst_hbm.at[c, t, 0], ib1d, semi1d).wait()

        @pl.when(j0 + 3 < CH)
        def _():
            pltpu.async_copy(src_hbm.at[c, t, j0 + 3], ib1s, semi1s)

        pltpu.sync_copy(rows1, acc_sh.at[ib1d], add=True)

        @pl.when(j0 + 3 < CH)
        def _():
            pltpu.async_copy(dst_hbm.at[c, t, j0 + 3], ib1d, semi1d)
            pltpu.make_async_copy(src_hbm.at[c, t, 0], ib1s, semi1s).wait()
            pltpu.async_copy(g_hbm.at[ib1s], rows1, sem1)

        return carry

    lax.fori_loop(0, CH // 2, pair, 0)

    plsc.subcore_barrier()
    pltpu.sync_copy(
        acc_sh.at[pl.ds(t * RPT, RPT)],
        out_hbm.at[c, pl.ds(t * RPT, RPT)],
    )


_RB = 400
_GRID = N_NODES // _RB


def _tc_g1_body(x_ref, w_ref, d0_ref, d1_ref, g_ref, dinv_ref):
    deg = d0_ref[...] + d1_ref[...] + 1.0
    dv = lax.rsqrt(deg)
    dinv_ref[...] = dv
    h = jnp.dot(x_ref[...], w_ref[...], preferred_element_type=jnp.float32)
    g_ref[...] = h * dv


def _tc_g1(x, W1, deg0, deg1):
    return pl.pallas_call(
        _tc_g1_body,
        grid=(_GRID,),
        in_specs=[
            pl.BlockSpec((_RB, D_FEAT), lambda i: (i, 0)),
            pl.BlockSpec((D_FEAT, N_HID), lambda i: (0, 0)),
            pl.BlockSpec((_RB, 1), lambda i: (i, 0)),
            pl.BlockSpec((_RB, 1), lambda i: (i, 0)),
        ],
        out_specs=[
            pl.BlockSpec((_RB, N_HID), lambda i: (i, 0)),
            pl.BlockSpec((_RB, 1), lambda i: (i, 0)),
        ],
        out_shape=[
            jax.ShapeDtypeStruct((N_NODES, N_HID), jnp.float32),
            jax.ShapeDtypeStruct((N_NODES, 1), jnp.float32),
        ],
    )(x, W1, deg0, deg1)


def _tc_mid_body(s_ref, g_ref, dv_ref, b_ref, w_ref, g2_ref):
    dv = dv_ref[...]
    z = jnp.maximum((s_ref[0] + s_ref[1] + g_ref[...]) * dv + b_ref[...], 0.0)
    h2 = jnp.dot(z, w_ref[...], preferred_element_type=jnp.float32)
    g2_ref[...] = h2 * dv


def _tc_mid(s, g1, dinv, b1, W2):
    return pl.pallas_call(
        _tc_mid_body,
        grid=(_GRID,),
        in_specs=[
            pl.BlockSpec((NC, _RB, N_HID), lambda i: (0, i, 0)),
            pl.BlockSpec((_RB, N_HID), lambda i: (i, 0)),
            pl.BlockSpec((_RB, 1), lambda i: (i, 0)),
            pl.BlockSpec((1, N_HID), lambda i: (0, 0)),
            pl.BlockSpec((N_HID, N_HID), lambda i: (0, 0)),
        ],
        out_specs=pl.BlockSpec((_RB, N_HID), lambda i: (i, 0)),
        out_shape=jax.ShapeDtypeStruct((N_NODES, N_HID), jnp.float32),
    )(s, g1, dinv, b1, W2)


def _tc_out_body(s_ref, g_ref, dv_ref, b_ref, o_ref):
    o_ref[...] = jnp.maximum(
        (s_ref[0] + s_ref[1] + g_ref[...]) * dv_ref[...] + b_ref[...], 0.0
    )


def _tc_out(s, g2, dinv, b2):
    return pl.pallas_call(
        _tc_out_body,
        grid=(_GRID,),
        in_specs=[
            pl.BlockSpec((NC, _RB, N_HID), lambda i: (0, i, 0)),
            pl.BlockSpec((_RB, N_HID), lambda i: (i, 0)),
            pl.BlockSpec((_RB, 1), lambda i: (i, 0)),
            pl.BlockSpec((1, N_HID), lambda i: (0, 0)),
        ],
        out_specs=pl.BlockSpec((_RB, N_HID), lambda i: (i, 0)),
        out_shape=jax.ShapeDtypeStruct((N_NODES, N_HID), jnp.float32),
    )(s, g2, dinv, b2)


@jax.jit
def kernel(x, edge_index, W1, b1, W2, b2):
    npad = EPAD - N_EDGES
    pad_iota = lax.iota(jnp.int32, npad)
    src = jnp.concatenate(
        [edge_index[0].astype(jnp.int32), pad_iota % N_NODES]
    ).reshape(NC, NS, CH, K)
    dst = jnp.concatenate(
        [edge_index[1].astype(jnp.int32), N_NODES + pad_iota % (NROWS - N_NODES)]
    ).reshape(NC, NS, CH, K)

    degp = _sc_deg(dst).reshape(NC, NPAD)
    deg0 = degp[0, :N_NODES].reshape(N_NODES, 1)
    deg1 = degp[1, :N_NODES].reshape(N_NODES, 1)

    g1, dinv = _tc_g1(x, W1, deg0, deg1)

    s = _sc_scatter(src, dst, g1)
    g2 = _tc_mid(s, g1, dinv, b1.reshape(1, N_HID), W2)

    s2 = _sc_scatter(src, dst, g2)
    return _tc_out(s2, g2, dinv, b2.reshape(1, N_HID))
